# Initial kernel scaffold; baseline (speedup 1.0000x reference)
import jax
import jax.numpy as jnp
from jax.experimental import pallas as pl


def kernel(x, edge_index, W1, b1, W2, b2):
    raise NotImplementedError("write your pallas kernel here")


# speedup vs baseline: 16.7873x; 16.7873x over previous
import functools

import jax
import jax.numpy as jnp
from jax import lax
from jax.experimental import pallas as pl
from jax.experimental.pallas import tpu as pltpu
from jax.experimental.pallas import tpu_sc as plsc

N_NODES = 10000
NC, NS, L = 2, 16, 16
NW = NC * NS
CHUNK = 128
ROWS_PER_TILE = 632
NPAD = NS * ROWS_PER_TILE

_MESH = plsc.VectorSubcoreMesh(
    core_axis_name="c", subcore_axis_name="s", num_cores=NC, num_subcores=NS
)
_SC_PARAMS = pltpu.CompilerParams(
    needs_layout_passes=False, use_tc_tiling_on_sc=False
)


def _deg_body(n_iter, dst_flat, degp, deg_v, didx_v):
    c = lax.axis_index("c")
    s = lax.axis_index("s")
    wid = c * NS + s

    def zero(r, carry):
        deg_v[pl.ds(r * L, L)] = jnp.zeros((L,), jnp.float32)
        return carry

    lax.fori_loop(0, NPAD // L, zero, 0)

    ones = jnp.ones((L,), jnp.float32)

    def body(j, carry):
        off = pl.multiple_of((wid * n_iter + j) * CHUNK, CHUNK)
        pltpu.sync_copy(dst_flat.at[pl.ds(off, CHUNK)], didx_v)
        for k in range(CHUNK // L):
            idx = didx_v[pl.ds(k * L, L)]
            plsc.addupdate_scatter(deg_v, [idx], ones)
        return carry

    lax.fori_loop(0, n_iter, body, 0)
    out_off = pl.multiple_of(wid * NPAD, 8)
    pltpu.sync_copy(deg_v, degp.at[pl.ds(out_off, NPAD)])


def _make_deg_call(n_iter):
    return pl.kernel(
        functools.partial(_deg_body, n_iter),
        out_type=jax.ShapeDtypeStruct((NW * NPAD,), jnp.float32),
        mesh=_MESH,
        scratch_types=[
            pltpu.VMEM((NPAD,), jnp.float32),
            pltpu.VMEM((CHUNK,), jnp.int32),
        ],
        compiler_params=_SC_PARAMS,
    )


def _agg_body(n_iter, feat, g_hbm, src_flat, dst_flat, out_hbm,
              acc_sh, sidx_v, didx_v, rows_v, zbuf, sem):
    c = lax.axis_index("c")
    s = lax.axis_index("s")
    wid = c * NS + s

    def zero(r, carry):
        for k in range(feat // L):
            zbuf[r, pl.ds(k * L, L)] = jnp.zeros((L,), jnp.float32)
        return carry

    lax.fori_loop(0, ROWS_PER_TILE, zero, 0)
    slab = pl.multiple_of(s * ROWS_PER_TILE, 8)
    pltpu.sync_copy(zbuf, acc_sh.at[pl.ds(slab, ROWS_PER_TILE)])
    plsc.subcore_barrier()

    def body(j, carry):
        off = pl.multiple_of((wid * n_iter + j) * CHUNK, CHUNK)
        pltpu.sync_copy(src_flat.at[pl.ds(off, CHUNK)], sidx_v)
        pltpu.sync_copy(dst_flat.at[pl.ds(off, CHUNK)], didx_v)
        pltpu.async_copy(g_hbm.at[sidx_v], rows_v, sem).wait()
        pltpu.sync_copy(rows_v, acc_sh.at[didx_v], add=True)
        return carry

    lax.fori_loop(0, n_iter, body, 0)
    plsc.subcore_barrier()

    pltpu.sync_copy(acc_sh.at[pl.ds(slab, ROWS_PER_TILE)], out_hbm.at[c, s])


def _make_agg_call(n_iter, feat):
    return pl.kernel(
        functools.partial(_agg_body, n_iter, feat),
        out_type=jax.ShapeDtypeStruct((NC, NS, ROWS_PER_TILE, feat), jnp.float32),
        mesh=_MESH,
        scratch_types=[
            pltpu.VMEM_SHARED((NPAD, feat), jnp.float32),
            pltpu.VMEM((CHUNK,), jnp.int32),
            pltpu.VMEM((CHUNK,), jnp.int32),
            pltpu.VMEM((CHUNK, feat), jnp.float32),
            pltpu.VMEM((ROWS_PER_TILE, feat), jnp.float32),
            pltpu.SemaphoreType.DMA,
        ],
        compiler_params=_SC_PARAMS,
    )


def _tc1_body(x_ref, w1_ref, degt_ref, g1_ref, dinv_ref):
    deg = jnp.sum(degt_ref[...], axis=1, keepdims=True) + 1.0
    dinv = lax.rsqrt(deg)
    h = jnp.dot(x_ref[...], w1_ref[...], preferred_element_type=jnp.float32)
    g1_ref[...] = h * dinv
    dinv_ref[...] = dinv


def _tc2_body(agg_ref, g1_ref, dinv_ref, b1_ref, w2_ref, g2_ref):
    y = agg_ref[0] + agg_ref[1] + g1_ref[...]
    t = jnp.maximum(dinv_ref[...] * y + b1_ref[...], 0.0)
    h2 = jnp.dot(t, w2_ref[...], preferred_element_type=jnp.float32)
    g2_ref[...] = h2 * dinv_ref[...]


def _tc3_body(agg_ref, g2_ref, dinv_ref, b2_ref, z_ref):
    z = dinv_ref[...] * (agg_ref[0] + agg_ref[1] + g2_ref[...]) + b2_ref[...]
    z_ref[...] = z


def kernel(x, edge_index, W1, b1, W2, b2):
    n_edges = edge_index.shape[1]
    hidden = W1.shape[1]
    z_dim = W2.shape[1]
    n_iter = -(-n_edges // (NW * CHUNK))
    e_pad = NW * n_iter * CHUNK

    src = edge_index[0]
    dst = edge_index[1]
    pad_s = jnp.zeros((e_pad - n_edges,), jnp.int32)
    pad_d = jnp.full((e_pad - n_edges,), N_NODES, jnp.int32)
    src_flat = jnp.concatenate([src, pad_s])
    dst_flat = jnp.concatenate([dst, pad_d])

    degp = _make_deg_call(n_iter)(dst_flat)
    degt = degp.reshape(NW, NPAD)[:, :N_NODES].T

    g1, dinv = pl.pallas_call(
        _tc1_body,
        out_shape=(
            jax.ShapeDtypeStruct((N_NODES, hidden), jnp.float32),
            jax.ShapeDtypeStruct((N_NODES, 1), jnp.float32),
        ),
    )(x, W1, degt)

    agg1 = _make_agg_call(n_iter, hidden)(g1, src_flat, dst_flat)
    agg1 = agg1.reshape(NC, NPAD, hidden)[:, :N_NODES]

    g2 = pl.pallas_call(
        _tc2_body,
        out_shape=jax.ShapeDtypeStruct((N_NODES, z_dim), jnp.float32),
    )(agg1, g1, dinv, b1.reshape(1, hidden), W2)

    agg2 = _make_agg_call(n_iter, z_dim)(g2, src_flat, dst_flat)
    agg2 = agg2.reshape(NC, NPAD, z_dim)[:, :N_NODES]

    z = pl.pallas_call(
        _tc3_body,
        out_shape=jax.ShapeDtypeStruct((N_NODES, z_dim), jnp.float32),
    )(agg2, g2, dinv, b2.reshape(1, z_dim))

    return z


# speedup vs baseline: 22.9755x; 1.3686x over previous
import functools

import jax
import jax.numpy as jnp
from jax import lax
from jax.experimental import pallas as pl
from jax.experimental.pallas import tpu as pltpu
from jax.experimental.pallas import tpu_sc as plsc

N_NODES = 10000
NC, NS, L = 2, 16, 16
NW = NC * NS
CHUNK = 128
NB = 4
ROWS_PER_TILE = 632
NPAD = NS * ROWS_PER_TILE

_MESH = plsc.VectorSubcoreMesh(
    core_axis_name="c", subcore_axis_name="s", num_cores=NC, num_subcores=NS
)
_SC_PARAMS = pltpu.CompilerParams(
    needs_layout_passes=False, use_tc_tiling_on_sc=False
)


def _deg_body(n_chunk, dst2, degp, deg_v, didx2, si):
    c_ax = lax.axis_index("c")
    s_ax = lax.axis_index("s")
    wid = c_ax * NS + s_ax

    islab = pl.multiple_of(wid * n_chunk, 8)
    cp = pltpu.async_copy(dst2.at[pl.ds(islab, n_chunk)], didx2, si)

    def zero(r, carry):
        deg_v[pl.ds(r * L, L)] = jnp.zeros((L,), jnp.float32)
        return carry

    lax.fori_loop(0, NPAD // L, zero, 0)
    cp.wait()

    ones = jnp.ones((L,), jnp.float32)

    def body(c, carry):
        for k in range(CHUNK // L):
            idx = didx2[c, pl.ds(k * L, L)]
            plsc.addupdate_scatter(deg_v, [idx], ones)
        return carry

    lax.fori_loop(0, n_chunk, body, 0)
    out_off = pl.multiple_of(wid * NPAD, 8)
    pltpu.sync_copy(deg_v, degp.at[pl.ds(out_off, NPAD)])


def _make_deg_call(n_chunk):
    return pl.kernel(
        functools.partial(_deg_body, n_chunk),
        out_type=jax.ShapeDtypeStruct((NW * NPAD,), jnp.float32),
        mesh=_MESH,
        scratch_types=[
            pltpu.VMEM((NPAD,), jnp.float32),
            pltpu.VMEM((n_chunk, CHUNK), jnp.int32),
            pltpu.SemaphoreType.DMA,
        ],
        compiler_params=_SC_PARAMS,
    )


def _agg_body(n_chunk, feat, g_hbm, src2, dst2, out_hbm,
              acc_sh, sidx2, didx2, rows0, rows1, rows2, rows3,
              si, sg0, sg1, sg2, sg3, ss0, ss1, ss2, ss3):
    rows = (rows0, rows1, rows2, rows3)
    sg = (sg0, sg1, sg2, sg3)
    ss = (ss0, ss1, ss2, ss3)
    c_ax = lax.axis_index("c")
    s_ax = lax.axis_index("s")
    wid = c_ax * NS + s_ax

    islab = pl.multiple_of(wid * n_chunk, 8)
    cp_s = pltpu.async_copy(src2.at[pl.ds(islab, n_chunk)], sidx2, si)
    cp_d = pltpu.async_copy(dst2.at[pl.ds(islab, n_chunk)], didx2, si)

    def zero(r, carry):
        for k in range(feat // L):
            rows0[r, pl.ds(k * L, L)] = jnp.zeros((L,), jnp.float32)
        return carry

    lax.fori_loop(0, CHUNK, zero, 0)
    slab = pl.multiple_of(s_ax * ROWS_PER_TILE, 8)
    n_full, rem = ROWS_PER_TILE // CHUNK, ROWS_PER_TILE % CHUNK
    for q in range(n_full):
        pltpu.sync_copy(rows0, acc_sh.at[pl.ds(slab + q * CHUNK, CHUNK)])
    if rem:
        pltpu.sync_copy(
            rows0.at[pl.ds(0, rem)],
            acc_sh.at[pl.ds(slab + n_full * CHUNK, rem)],
        )
    cp_s.wait()
    cp_d.wait()
    plsc.subcore_barrier()

    def gather_start(c, b):
        pltpu.async_copy(g_hbm.at[sidx2.at[c]], rows[b], sg[b])

    def gather_wait(b):
        pltpu.make_async_copy(g_hbm.at[sidx2.at[0]], rows[b], sg[b]).wait()

    def scatter_start(c, b):
        pltpu.async_copy(rows[b], acc_sh.at[didx2.at[c]], ss[b], add=True)

    def scatter_wait(b):
        pltpu.make_async_copy(rows[b], acc_sh.at[pl.ds(0, CHUNK)], ss[b]).wait()

    def outer(t, carry):
        for b in range(NB):
            c = t * NB + b
            pb = (b + NB - 1) % NB

            @pl.when(t > 0)
            def _():
                scatter_wait(b)

            gather_start(c, b)
            if b == 0:
                @pl.when(t > 0)
                def _():
                    gather_wait(pb)
                    scatter_start(t * NB - 1, pb)
            else:
                gather_wait(pb)
                scatter_start(c - 1, pb)
        return carry

    lax.fori_loop(0, n_chunk // NB, outer, 0)

    gather_wait(NB - 1)
    scatter_start(n_chunk - 1, NB - 1)
    for b in range(NB):
        scatter_wait(b)
    plsc.subcore_barrier()
    pltpu.sync_copy(acc_sh.at[pl.ds(slab, ROWS_PER_TILE)], out_hbm.at[c_ax, s_ax])


def _make_agg_call(n_chunk, feat):
    return pl.kernel(
        functools.partial(_agg_body, n_chunk, feat),
        out_type=jax.ShapeDtypeStruct((NC, NS, ROWS_PER_TILE, feat), jnp.float32),
        mesh=_MESH,
        scratch_types=[
            pltpu.VMEM_SHARED((NPAD, feat), jnp.float32),
            pltpu.VMEM((n_chunk, CHUNK), jnp.int32),
            pltpu.VMEM((n_chunk, CHUNK), jnp.int32),
            pltpu.VMEM((CHUNK, feat), jnp.float32),
            pltpu.VMEM((CHUNK, feat), jnp.float32),
            pltpu.VMEM((CHUNK, feat), jnp.float32),
            pltpu.VMEM((CHUNK, feat), jnp.float32),
        ] + [pltpu.SemaphoreType.DMA] * 9,
        compiler_params=_SC_PARAMS,
    )


def _tc1_body(x_ref, w1_ref, degt_ref, g1_ref, dinv_ref):
    deg = jnp.sum(degt_ref[...], axis=1, keepdims=True) + 1.0
    dinv = lax.rsqrt(deg)
    h = jnp.dot(x_ref[...], w1_ref[...], preferred_element_type=jnp.float32)
    g1_ref[...] = h * dinv
    dinv_ref[...] = dinv


def _tc2_body(agg_ref, g1_ref, dinv_ref, b1_ref, w2_ref, g2_ref):
    y = agg_ref[0] + agg_ref[1] + g1_ref[...]
    t = jnp.maximum(dinv_ref[...] * y + b1_ref[...], 0.0)
    h2 = jnp.dot(t, w2_ref[...], preferred_element_type=jnp.float32)
    g2_ref[...] = h2 * dinv_ref[...]


def _tc3_body(agg_ref, g2_ref, dinv_ref, b2_ref, z_ref):
    z = dinv_ref[...] * (agg_ref[0] + agg_ref[1] + g2_ref[...]) + b2_ref[...]
    z_ref[...] = z


def kernel(x, edge_index, W1, b1, W2, b2):
    n_edges = edge_index.shape[1]
    hidden = W1.shape[1]
    z_dim = W2.shape[1]
    n_chunk = -(-n_edges // (NW * CHUNK * NB)) * NB
    e_pad = NW * n_chunk * CHUNK

    src = edge_index[0]
    dst = edge_index[1]
    pad_s = jnp.zeros((e_pad - n_edges,), jnp.int32)
    pad_d = jnp.full((e_pad - n_edges,), N_NODES, jnp.int32)
    src2 = jnp.concatenate([src, pad_s]).reshape(NW * n_chunk, CHUNK)
    dst2 = jnp.concatenate([dst, pad_d]).reshape(NW * n_chunk, CHUNK)

    degp = _make_deg_call(n_chunk)(dst2)
    degt = degp.reshape(NW, NPAD)[:, :N_NODES].T

    g1, dinv = pl.pallas_call(
        _tc1_body,
        out_shape=(
            jax.ShapeDtypeStruct((N_NODES, hidden), jnp.float32),
            jax.ShapeDtypeStruct((N_NODES, 1), jnp.float32),
        ),
    )(x, W1, degt)

    agg1 = _make_agg_call(n_chunk, hidden)(g1, src2, dst2)
    agg1 = agg1.reshape(NC, NPAD, hidden)[:, :N_NODES]

    g2 = pl.pallas_call(
        _tc2_body,
        out_shape=jax.ShapeDtypeStruct((N_NODES, z_dim), jnp.float32),
    )(agg1, g1, dinv, b1.reshape(1, hidden), W2)

    agg2 = _make_agg_call(n_chunk, z_dim)(g2, src2, dst2)
    agg2 = agg2.reshape(NC, NPAD, z_dim)[:, :N_NODES]

    z = pl.pallas_call(
        _tc3_body,
        out_shape=jax.ShapeDtypeStruct((N_NODES, z_dim), jnp.float32),
    )(agg2, g2, dinv, b2.reshape(1, z_dim))

    return z
